# baseline (speedup 1.0000x reference)
import functools

import jax
import jax.numpy as jnp
from jax import lax
from jax.experimental import pallas as pl
from jax.experimental.pallas import tpu as pltpu
from jax.experimental.pallas import tpu_sc as plsc

_ROWS = 16906
_DIM = 200
_NC = 2
_NS = 16
_NW = _NC * _NS
_CHUNK = 528
_SUB = 176
_NSUB = _CHUNK // _SUB
_NBUF = 2
_TAIL = _NW * _CHUNK


@functools.partial(
    pl.kernel,
    out_type=jax.ShapeDtypeStruct((_ROWS, _DIM), jnp.float32),
    mesh=plsc.VectorSubcoreMesh(core_axis_name="c", subcore_axis_name="s"),
    compiler_params=pltpu.CompilerParams(
        skip_device_barrier=True,
        disable_bounds_checks=True,
        disable_semaphore_checks=True,
    ),
    scratch_types=[
        pltpu.VMEM((_NBUF, _SUB, _DIM), jnp.float32),
        pltpu.SemaphoreType.DMA((_NSUB,)),
        pltpu.SemaphoreType.DMA((_NSUB,)),
    ],
)
def _pos_embed_copy(table_hbm, out_hbm, bufs, in_sems, out_sems):
    wid = lax.axis_index("s") * _NC + lax.axis_index("c")
    base = wid * _CHUNK

    def in_copy(j):
        return pltpu.make_async_copy(
            table_hbm.at[pl.ds(base + j * _SUB, _SUB), :],
            bufs.at[j % _NBUF],
            in_sems.at[j],
        )

    def out_copy(j):
        return pltpu.make_async_copy(
            bufs.at[j % _NBUF],
            out_hbm.at[pl.ds(base + j * _SUB, _SUB), :],
            out_sems.at[j],
        )

    for j in range(min(_NBUF, _NSUB)):
        in_copy(j).start()
    for j in range(_NSUB):
        in_copy(j).wait()
        out_copy(j).start()
        nxt = j + 1
        if _NBUF <= nxt < _NSUB:
            out_copy(nxt - _NBUF).wait()
            in_copy(nxt).start()

    @pl.when(wid == 0)
    def _tail_a():
        pltpu.sync_copy(
            table_hbm.at[pl.ds(_TAIL, 8), :],
            out_hbm.at[pl.ds(_TAIL, 8), :],
        )

    @pl.when(wid == 1)
    def _tail_b():
        pltpu.sync_copy(
            table_hbm.at[pl.ds(_TAIL + 8, 2), :],
            out_hbm.at[pl.ds(_TAIL + 8, 2), :],
        )

    for j in range(_NSUB):
        if j + _NBUF >= _NSUB:
            out_copy(j).wait()


def kernel(x, table):
    del x
    return _pos_embed_copy(table)
